# baseline (speedup 1.0000x reference)
import functools

import jax
import jax.numpy as jnp
from jax import lax
from jax.experimental import pallas as pl
from jax.experimental.pallas import tpu as pltpu
from jax.experimental.pallas import tpu_sc as plsc


def _sc_gather(table, idx):
    V, D = table.shape
    B = idx.shape[0]
    info = plsc.get_sparse_core_info()
    num_workers = info.num_cores * info.num_subcores
    b_per_w = B // num_workers
    mesh = plsc.VectorSubcoreMesh(core_axis_name="c", subcore_axis_name="s")

    @functools.partial(
        pl.kernel,
        mesh=mesh,
        out_type=jax.ShapeDtypeStruct((B, D), jnp.float32),
        compiler_params=pltpu.CompilerParams(use_tc_tiling_on_sc=False),
        scratch_types=[
            pltpu.VMEM((b_per_w,), jnp.int32),
            pltpu.VMEM((b_per_w, D), jnp.float32),
            pltpu.SemaphoreType.DMA,
        ],
    )
    def gather_kernel(table_hbm, idx_hbm, out_hbm, idx_v, rows_v, sem):
        wid = lax.axis_index("s") * info.num_cores + lax.axis_index("c")
        base = wid * b_per_w
        pltpu.sync_copy(idx_hbm.at[pl.ds(base, b_per_w)], idx_v)
        pltpu.async_copy(table_hbm.at[idx_v], rows_v, sem).wait()
        pltpu.sync_copy(rows_v, out_hbm.at[pl.ds(base, b_per_w)])

    return gather_kernel(table, idx)


def _broadcast_body(lat_ref, wg_ref, wov_ref, o1_ref, o2_ref):
    lat = lat_ref[...]
    o1_ref[...] = lat * wg_ref[...]
    o2_ref[...] = lat * wov_ref[...]


def kernel(latent, genes_oi, W_height, W_overall):
    C = latent.shape[0]
    G = genes_oi.shape[0]
    V, D = W_height.shape
    N = W_overall.shape[0]

    wg = _sc_gather(W_height, genes_oi.astype(jnp.int32))

    lat2 = latent.reshape(C, 1)
    wgf = wg.reshape(1, G * D)
    wovf = W_overall.reshape(1, N)

    CB = 8
    out1, out2 = pl.pallas_call(
        _broadcast_body,
        grid=(C // CB,),
        in_specs=[
            pl.BlockSpec((CB, 1), lambda i: (i, 0)),
            pl.BlockSpec((1, G * D), lambda i: (0, 0)),
            pl.BlockSpec((1, N), lambda i: (0, 0)),
        ],
        out_specs=[
            pl.BlockSpec((CB, G * D), lambda i: (i, 0)),
            pl.BlockSpec((CB, N), lambda i: (i, 0)),
        ],
        out_shape=[
            jax.ShapeDtypeStruct((C, G * D), jnp.float32),
            jax.ShapeDtypeStruct((C, N), jnp.float32),
        ],
    )(lat2, wgf, wovf)

    return out1.reshape(C, G, D), out2.reshape(C, N, 1)
